# baseline (speedup 1.0000x reference)
import functools

import jax
import jax.numpy as jnp
from jax import lax
from jax.experimental import pallas as pl
from jax.experimental.pallas import tpu as pltpu
from jax.experimental.pallas import tpu_sc as plsc

N = 10000
E = 320000
F = 128
DELTA = 2.5

NSUB = 16
RNG = 320
CRNG = 5120
NPAD = 32 * RNG
HAF = 64
AW = 128
SCH = 640
NCH = E // SCH
GC = 32
LCAP = 13056
MMW = 20608
DGW = 5248
NEG = -3.0e38
POS = 3.0e38


def _fire_scan(dst_hbm, src_hbm, dv, sv, semd, sems, k):
    pltpu.async_copy(dst_hbm.at[pl.ds(k * SCH, SCH)], dv, semd)
    pltpu.async_copy(src_hbm.at[pl.ds(k * SCH, SCH)], sv, sems)


def _wait_scan(dst_hbm, src_hbm, dv, sv, semd, sems):
    pltpu.make_async_copy(dst_hbm.at[pl.ds(0, SCH)], dv, semd).wait()
    pltpu.make_async_copy(src_hbm.at[pl.ds(0, SCH)], sv, sems).wait()


def _sc_body(dst_hbm, src_hbm, xaug_hbm, sums_hbm, mxo_hbm, mno_hbm, deg_hbm,
             sums_shared, accmax, accmin, degacc, lst, dA, dB, sA, sB,
             mbA, mbB, gA, gB, iA, iB, lA, lB, semdA, semdB, semsA, semsB,
             semgA, semgB, semcA, semcB):
    c = lax.axis_index("c")
    s = lax.axis_index("s")
    q = c * NSUB + s
    base = q * RNG
    sbase = s * RNG
    onehot = jnp.where(lax.iota(jnp.int32, 16) == 0, 1.0, 0.0)

    def zero_mb(r, _):
        for k in range(AW // 16):
            mbA[r, pl.ds(k * 16, 16)] = jnp.zeros((16,), jnp.float32)
        return 0
    lax.fori_loop(0, GC, zero_mb, 0)

    def compact(dv, sv, off):
        for g in range(SCH // 16):
            d16 = dv[pl.ds(g * 16, 16)]
            s16 = sv[pl.ds(g * 16, 16)]
            t = d16 - base
            msk = (t >= 0) & (t < RNG)
            pk = (t << 14) | s16
            ci = plsc.cumsum(msk.astype(jnp.int32))
            pos = off + ci - 1
            plsc.store_scatter(lst, [pos], pk, mask=msk)
            cnt = plsc.all_reduce_population_count(msk)
            off = jnp.minimum(off + cnt[0], LCAP - 16)
        return off

    _fire_scan(dst_hbm, src_hbm, dA, sA, semdA, semsA, 0)

    def scan_body(cc, off):
        _wait_scan(dst_hbm, src_hbm, dA, sA, semdA, semsA)
        _fire_scan(dst_hbm, src_hbm, dB, sB, semdB, semsB,
                   jnp.minimum(2 * cc + 1, NCH - 1))
        off = compact(dA, sA, off)
        _wait_scan(dst_hbm, src_hbm, dB, sB, semdB, semsB)
        _fire_scan(dst_hbm, src_hbm, dA, sA, semdA, semsA,
                   jnp.minimum(2 * cc + 2, NCH - 1))
        off = compact(dB, sB, off)
        return off

    ne = lax.fori_loop(0, NCH // 2, scan_body, jnp.int32(0))
    _wait_scan(dst_hbm, src_hbm, dA, sA, semdA, semsA)

    ng = (ne + GC - 1) // GC
    kmax = jnp.maximum(ng - 1, 0)
    npairs = (ng + 1) // 2

    for p in (0, 1):
        def init_mm(i, _):
            accmax[pl.ds(i * 16, 16)] = jnp.full((16,), NEG, jnp.float32)
            accmin[pl.ds(i * 16, 16)] = jnp.full((16,), POS, jnp.float32)
            return 0
        lax.fori_loop(0, MMW // 16, init_mm, 0)

        if p == 0:
            def init_dg(i, _):
                degacc[pl.ds(i * 16, 16)] = jnp.zeros((16,), jnp.float32)
                return 0
            lax.fori_loop(0, DGW // 16, init_dg, 0)

        for k in range(RNG // GC):
            pltpu.sync_copy(mbA, sums_shared.at[pl.ds(sbase + k * GC, GC)])

        @pl.when(s == 0)
        def _zero_trash():
            pltpu.sync_copy(mbA.at[pl.ds(0, 8)],
                            sums_shared.at[pl.ds(CRNG, 8)])

        def fire_gather(gi, si, li, mb, semg, k):
            for g in range(GC // 16):
                pkv = lst[pl.ds(k * GC + g * 16, 16)]
                lane = k * GC + g * 16 + lax.iota(jnp.int32, 16)
                padm = lane >= ne
                t = pkv >> 14
                srcv = pkv & 0x3FFF
                gi[pl.ds(g * 16, 16)] = jnp.where(padm, 0, srcv) + p * N
                si[pl.ds(g * 16, 16)] = jnp.where(padm, CRNG, sbase + t)
                li[pl.ds(g * 16, 16)] = jnp.where(padm, RNG, t)
            pltpu.async_copy(xaug_hbm.at[gi], mb, semg)

        def wait_gather(gi, mb, semg):
            pltpu.make_async_copy(xaug_hbm.at[gi], mb, semg).wait()

        def process(mb, si, li, semc):
            pltpu.async_copy(mb, sums_shared.at[si], semc, add=True)
            for g in range(GC // 16):
                lv = li[pl.ds(g * 16, 16)]
                for l in range(16):
                    t = lv[l]
                    tb = t * HAF
                    for kq in range(4):
                        fsl = pl.ds(tb + kq * 16, 16)
                        msg = mb[g * 16 + l, pl.ds(kq * 16, 16)]
                        accmax[fsl] = jnp.maximum(accmax[fsl], msg)
                        accmin[fsl] = jnp.minimum(accmin[fsl], msg)
                    if p == 0:
                        dsl = pl.ds(t * 16, 16)
                        degacc[dsl] = degacc[dsl] + onehot
            pltpu.make_async_copy(mb, sums_shared.at[si], semc).wait()

        @pl.when(ng > 0)
        def _prologue():
            fire_gather(gA, iA, lA, mbA, semgA, jnp.int32(0))

        def gather_body(j, _):
            wait_gather(gA, mbA, semgA)
            fire_gather(gB, iB, lB, mbB, semgB, jnp.minimum(2 * j + 1, kmax))
            process(mbA, iA, lA, semcA)

            @pl.when(2 * j + 1 < ng)
            def _():
                wait_gather(gB, mbB, semgB)
                fire_gather(gA, iA, lA, mbA, semgA,
                            jnp.minimum(2 * j + 2, kmax))
                process(mbB, iB, lB, semcB)
            return 0

        lax.fori_loop(0, npairs, gather_body, 0)

        @pl.when(ng > 0)
        def _drain():
            @pl.when((ng % 2) == 1)
            def _():
                wait_gather(gB, mbB, semgB)

            @pl.when((ng % 2) == 0)
            def _():
                wait_gather(gA, mbA, semgA)

        pltpu.sync_copy(sums_shared.at[pl.ds(sbase, RNG)],
                        sums_hbm.at[pl.ds(p * NPAD + base, RNG)])
        mmo = (p * NPAD + base) * HAF
        pltpu.sync_copy(accmax.at[pl.ds(0, RNG * HAF)],
                        mxo_hbm.at[pl.ds(mmo, RNG * HAF)])
        pltpu.sync_copy(accmin.at[pl.ds(0, RNG * HAF)],
                        mno_hbm.at[pl.ds(mmo, RNG * HAF)])

        if p == 0:
            pltpu.sync_copy(degacc.at[pl.ds(0, RNG * 16)],
                            deg_hbm.at[pl.ds(base * 16, RNG * 16)])
            lax.fori_loop(0, GC, zero_mb, 0)


def _sc_stats(dst, src, xaug):
    mesh = plsc.VectorSubcoreMesh(core_axis_name="c", subcore_axis_name="s")
    f = pl.kernel(
        _sc_body,
        out_type=(
            jax.ShapeDtypeStruct((2 * NPAD, AW), jnp.float32),
            jax.ShapeDtypeStruct((2 * NPAD * HAF,), jnp.float32),
            jax.ShapeDtypeStruct((2 * NPAD * HAF,), jnp.float32),
            jax.ShapeDtypeStruct((NPAD * 16,), jnp.float32),
        ),
        mesh=mesh,
        compiler_params=pltpu.CompilerParams(needs_layout_passes=False),
        scratch_types=[
            pltpu.VMEM_SHARED((CRNG + 8, AW), jnp.float32),
            pltpu.VMEM((MMW,), jnp.float32),
            pltpu.VMEM((MMW,), jnp.float32),
            pltpu.VMEM((DGW,), jnp.float32),
            pltpu.VMEM((LCAP,), jnp.int32),
            pltpu.VMEM((SCH,), jnp.int32),
            pltpu.VMEM((SCH,), jnp.int32),
            pltpu.VMEM((SCH,), jnp.int32),
            pltpu.VMEM((SCH,), jnp.int32),
            pltpu.VMEM((GC, AW), jnp.float32),
            pltpu.VMEM((GC, AW), jnp.float32),
            pltpu.VMEM((GC,), jnp.int32),
            pltpu.VMEM((GC,), jnp.int32),
            pltpu.VMEM((GC,), jnp.int32),
            pltpu.VMEM((GC,), jnp.int32),
            pltpu.VMEM((GC,), jnp.int32),
            pltpu.VMEM((GC,), jnp.int32),
            pltpu.SemaphoreType.DMA,
            pltpu.SemaphoreType.DMA,
            pltpu.SemaphoreType.DMA,
            pltpu.SemaphoreType.DMA,
            pltpu.SemaphoreType.DMA,
            pltpu.SemaphoreType.DMA,
            pltpu.SemaphoreType.DMA,
            pltpu.SemaphoreType.DMA,
        ],
    )
    return f(dst, src, xaug)


R = 1000
NB = N // R


def _dense_body(deg_ref, s_ref, ss_ref, mx_ref, mn_ref, x_ref,
                w1i_ref, w1a_ref, w1t_ref, b1_ref, w2_ref, b2_ref,
                w3_ref, b3_ref, g_ref, be_ref, out_ref, h3_scr, acc_scr):
    p = pl.program_id(0)
    i = pl.program_id(1)

    @pl.when(p == 0)
    def _phase0():
        deg = deg_ref[:, :]
        degs = jnp.maximum(deg, 1.0)
        has = deg > 0
        mean = s_ref[:, :] / degs
        meansq = ss_ref[:, :] / degs
        var = jnp.maximum(meansq - mean * mean, 0.0)
        std = jnp.sqrt(var + 1e-5)
        mx = jnp.where(has, mx_ref[:, :], 0.0)
        mn = jnp.where(has, mn_ref[:, :], 0.0)
        agg = jnp.concatenate([mean, mx, mn, std], axis=1)
        logd = jnp.log(deg + 1.0)
        ampf = logd / DELTA
        attf = jnp.where(deg > 0, DELTA / jnp.where(logd > 0, logd, 1.0), 0.0)
        hh = jnp.concatenate([agg, agg * ampf, agg * attf], axis=1)
        w1 = jnp.concatenate([w1i_ref[:, :], w1a_ref[:, :], w1t_ref[:, :]],
                             axis=0)
        h1 = jnp.maximum(
            jnp.dot(hh, w1, preferred_element_type=jnp.float32)
            + b1_ref[:, :], 0.0)
        h2 = jnp.maximum(
            jnp.dot(h1, w2_ref[:, :], preferred_element_type=jnp.float32)
            + b2_ref[:, :], 0.0)
        h3 = (jnp.dot(h2, w3_ref[:, :], preferred_element_type=jnp.float32)
              + b3_ref[:, :])
        h3_scr[pl.ds(i * R, R), :] = h3

        @pl.when(i == 0)
        def _init():
            acc_scr[:, :] = jnp.zeros_like(acc_scr)

        acc_scr[0:1, :] += jnp.sum(h3, axis=0, keepdims=True)
        acc_scr[1:2, :] += jnp.sum(h3 * h3, axis=0, keepdims=True)

    @pl.when(p == 1)
    def _phase1():
        mu = acc_scr[0:1, :] / float(N)
        var = acc_scr[1:2, :] / float(N) - mu * mu
        h3 = h3_scr[pl.ds(i * R, R), :]
        hn = (h3 - mu) / jnp.sqrt(var + 1e-5) * g_ref[:, :] + be_ref[:, :]
        out_ref[:, :] = jnp.maximum(hn, 0.0) + x_ref[:, :]


@functools.partial(jax.jit, static_argnames=("interpret",))
def _dense(deg, s, ss, mx, mn, x, W1, b1, W2, b2, W3, b3, gamma, beta,
           interpret=False):
    w1i = W1[0:512]
    w1a = W1[512:1024]
    w1t = W1[1024:1536]
    row = lambda r: pl.BlockSpec((R, r), lambda p, i: (i, 0))
    full = lambda a, b: pl.BlockSpec((a, b), lambda p, i: (0, 0))
    return pl.pallas_call(
        _dense_body,
        grid=(2, NB),
        in_specs=[
            row(1), row(F), row(F), row(F), row(F), row(F),
            full(512, F), full(512, F), full(512, F), full(1, F),
            full(F, F), full(1, F), full(F, F), full(1, F),
            full(1, F), full(1, F),
        ],
        out_specs=row(F),
        out_shape=jax.ShapeDtypeStruct((N, F), jnp.float32),
        scratch_shapes=[
            pltpu.VMEM((N, F), jnp.float32),
            pltpu.VMEM((2, F), jnp.float32),
        ],
        interpret=interpret,
    )(deg.reshape(N, 1), s, ss, mx, mn, x,
      w1i, w1a, w1t, b1.reshape(1, F), W2, b2.reshape(1, F),
      W3, b3.reshape(1, F), gamma.reshape(1, F), beta.reshape(1, F))


def kernel(x, edge_index, W1, b1, W2, b2, W3, b3, gamma, beta):
    src = edge_index[0]
    dst = edge_index[1]
    xc = jnp.stack([x[:, :HAF], x[:, HAF:]])
    xaug = jnp.concatenate([xc, xc * xc], axis=-1).reshape(2 * N, AW)

    sums, mxo, mno, dego = _sc_stats(dst, src, xaug)
    s = jnp.concatenate([sums[:N, :HAF], sums[NPAD:NPAD + N, :HAF]], axis=1)
    ss = jnp.concatenate([sums[:N, HAF:], sums[NPAD:NPAD + N, HAF:]], axis=1)
    mxr = mxo.reshape(2, NPAD, HAF)
    mnr = mno.reshape(2, NPAD, HAF)
    mx = jnp.concatenate([mxr[0, :N], mxr[1, :N]], axis=1)
    mn = jnp.concatenate([mnr[0, :N], mnr[1, :N]], axis=1)
    deg = dego.reshape(NPAD, 16)[:N, 0]
    return _dense(deg, s, ss, mx, mn, x, W1, b1, W2, b2, W3, b3, gamma, beta)


# speedup vs baseline: 1.0117x; 1.0117x over previous
import functools

import jax
import jax.numpy as jnp
from jax import lax
from jax.experimental import pallas as pl
from jax.experimental.pallas import tpu as pltpu
from jax.experimental.pallas import tpu_sc as plsc

N = 10000
E = 320000
F = 128
DELTA = 2.5

NSUB = 16
RNG = 320
CRNG = 5120
NPAD = 32 * RNG
HAF = 64
AW = 128
SCH = 640
NCH = E // SCH
GC = 32
LCAP = 13056
MMW = 20608
DGW = 5248
NEG = -3.0e38
POS = 3.0e38


def _fire_scan(dst_hbm, src_hbm, dv, sv, semd, sems, k):
    pltpu.async_copy(dst_hbm.at[pl.ds(k * SCH, SCH)], dv, semd)
    pltpu.async_copy(src_hbm.at[pl.ds(k * SCH, SCH)], sv, sems)


def _wait_scan(dst_hbm, src_hbm, dv, sv, semd, sems):
    pltpu.make_async_copy(dst_hbm.at[pl.ds(0, SCH)], dv, semd).wait()
    pltpu.make_async_copy(src_hbm.at[pl.ds(0, SCH)], sv, sems).wait()


def _sc_body(dst_hbm, src_hbm, xaug_hbm, sums_hbm, mxo_hbm, mno_hbm, deg_hbm,
             sums_shared, accmax, accmin, degacc, lst, dA, dB, sA, sB,
             mbA, mbB, gA, gB, iA, iB, lA, lB, semdA, semdB, semsA, semsB,
             semgA, semgB, semcA, semcB):
    c = lax.axis_index("c")
    s = lax.axis_index("s")
    q = c * NSUB + s
    base = q * RNG
    sbase = s * RNG
    onehot = jnp.where(lax.iota(jnp.int32, 16) == 0, 1.0, 0.0)

    def zero_mb(r, _):
        for k in range(AW // 16):
            mbA[r, pl.ds(k * 16, 16)] = jnp.zeros((16,), jnp.float32)
        return 0
    lax.fori_loop(0, GC, zero_mb, 0)

    def compact(dv, sv, off):
        for g in range(0, SCH // 16, 2):
            da = dv[pl.ds(g * 16, 16)]
            sa = sv[pl.ds(g * 16, 16)]
            db = dv[pl.ds(g * 16 + 16, 16)]
            sb = sv[pl.ds(g * 16 + 16, 16)]
            ta = da - base
            tb = db - base
            mska = (ta >= 0) & (ta < RNG)
            mskb = (tb >= 0) & (tb < RNG)
            cia = plsc.cumsum(mska.astype(jnp.int32))
            cib = plsc.cumsum(mskb.astype(jnp.int32))
            cnta = plsc.all_reduce_population_count(mska)
            cntb = plsc.all_reduce_population_count(mskb)
            pka = (ta << 14) | sa
            pkb = (tb << 14) | sb
            plsc.store_scatter(lst, [off + cia - 1], pka, mask=mska)
            offa = jnp.minimum(off + cnta[0], LCAP - 16)
            plsc.store_scatter(lst, [offa + cib - 1], pkb, mask=mskb)
            off = jnp.minimum(offa + cntb[0], LCAP - 16)
        return off

    _fire_scan(dst_hbm, src_hbm, dA, sA, semdA, semsA, 0)

    def scan_body(cc, off):
        _wait_scan(dst_hbm, src_hbm, dA, sA, semdA, semsA)
        _fire_scan(dst_hbm, src_hbm, dB, sB, semdB, semsB,
                   jnp.minimum(2 * cc + 1, NCH - 1))
        off = compact(dA, sA, off)
        _wait_scan(dst_hbm, src_hbm, dB, sB, semdB, semsB)
        _fire_scan(dst_hbm, src_hbm, dA, sA, semdA, semsA,
                   jnp.minimum(2 * cc + 2, NCH - 1))
        off = compact(dB, sB, off)
        return off

    ne = lax.fori_loop(0, NCH // 2, scan_body, jnp.int32(0))
    _wait_scan(dst_hbm, src_hbm, dA, sA, semdA, semsA)

    ng = (ne + GC - 1) // GC
    kmax = jnp.maximum(ng - 1, 0)
    npairs = (ng + 1) // 2

    for p in (0, 1):
        def init_mm(i, _):
            accmax[pl.ds(i * 16, 16)] = jnp.full((16,), NEG, jnp.float32)
            accmin[pl.ds(i * 16, 16)] = jnp.full((16,), POS, jnp.float32)
            return 0
        lax.fori_loop(0, MMW // 16, init_mm, 0)

        if p == 0:
            def init_dg(i, _):
                degacc[pl.ds(i * 16, 16)] = jnp.zeros((16,), jnp.float32)
                return 0
            lax.fori_loop(0, DGW // 16, init_dg, 0)

        for k in range(RNG // GC):
            pltpu.sync_copy(mbA, sums_shared.at[pl.ds(sbase + k * GC, GC)])

        @pl.when(s == 0)
        def _zero_trash():
            pltpu.sync_copy(mbA.at[pl.ds(0, 8)],
                            sums_shared.at[pl.ds(CRNG, 8)])

        def fire_gather(gi, si, li, mb, semg, k):
            for g in range(GC // 16):
                pkv = lst[pl.ds(k * GC + g * 16, 16)]
                lane = k * GC + g * 16 + lax.iota(jnp.int32, 16)
                padm = lane >= ne
                t = pkv >> 14
                srcv = pkv & 0x3FFF
                gi[pl.ds(g * 16, 16)] = jnp.where(padm, 0, srcv) + p * N
                si[pl.ds(g * 16, 16)] = jnp.where(padm, CRNG, sbase + t)
                li[pl.ds(g * 16, 16)] = jnp.where(padm, RNG, t)
            pltpu.async_copy(xaug_hbm.at[gi], mb, semg)

        def wait_gather(gi, mb, semg):
            pltpu.make_async_copy(xaug_hbm.at[gi], mb, semg).wait()

        def process(mb, si, li, semc):
            pltpu.async_copy(mb, sums_shared.at[si], semc, add=True)
            for g in range(GC // 16):
                lv = li[pl.ds(g * 16, 16)]
                for l in range(16):
                    t = lv[l]
                    tb = t * HAF
                    for kq in range(4):
                        fsl = pl.ds(tb + kq * 16, 16)
                        msg = mb[g * 16 + l, pl.ds(kq * 16, 16)]
                        accmax[fsl] = jnp.maximum(accmax[fsl], msg)
                        accmin[fsl] = jnp.minimum(accmin[fsl], msg)
                    if p == 0:
                        dsl = pl.ds(t * 16, 16)
                        degacc[dsl] = degacc[dsl] + onehot
            pltpu.make_async_copy(mb, sums_shared.at[si], semc).wait()

        @pl.when(ng > 0)
        def _prologue():
            fire_gather(gA, iA, lA, mbA, semgA, jnp.int32(0))

        def gather_body(j, _):
            wait_gather(gA, mbA, semgA)
            fire_gather(gB, iB, lB, mbB, semgB, jnp.minimum(2 * j + 1, kmax))
            process(mbA, iA, lA, semcA)

            @pl.when(2 * j + 1 < ng)
            def _():
                wait_gather(gB, mbB, semgB)
                fire_gather(gA, iA, lA, mbA, semgA,
                            jnp.minimum(2 * j + 2, kmax))
                process(mbB, iB, lB, semcB)
            return 0

        lax.fori_loop(0, npairs, gather_body, 0)

        @pl.when(ng > 0)
        def _drain():
            @pl.when((ng % 2) == 1)
            def _():
                wait_gather(gB, mbB, semgB)

            @pl.when((ng % 2) == 0)
            def _():
                wait_gather(gA, mbA, semgA)

        pltpu.sync_copy(sums_shared.at[pl.ds(sbase, RNG)],
                        sums_hbm.at[pl.ds(p * NPAD + base, RNG)])
        mmo = (p * NPAD + base) * HAF
        pltpu.sync_copy(accmax.at[pl.ds(0, RNG * HAF)],
                        mxo_hbm.at[pl.ds(mmo, RNG * HAF)])
        pltpu.sync_copy(accmin.at[pl.ds(0, RNG * HAF)],
                        mno_hbm.at[pl.ds(mmo, RNG * HAF)])

        if p == 0:
            pltpu.sync_copy(degacc.at[pl.ds(0, RNG * 16)],
                            deg_hbm.at[pl.ds(base * 16, RNG * 16)])
            lax.fori_loop(0, GC, zero_mb, 0)


def _sc_stats(dst, src, xaug):
    mesh = plsc.VectorSubcoreMesh(core_axis_name="c", subcore_axis_name="s")
    f = pl.kernel(
        _sc_body,
        out_type=(
            jax.ShapeDtypeStruct((2 * NPAD, AW), jnp.float32),
            jax.ShapeDtypeStruct((2 * NPAD * HAF,), jnp.float32),
            jax.ShapeDtypeStruct((2 * NPAD * HAF,), jnp.float32),
            jax.ShapeDtypeStruct((NPAD * 16,), jnp.float32),
        ),
        mesh=mesh,
        compiler_params=pltpu.CompilerParams(needs_layout_passes=False),
        scratch_types=[
            pltpu.VMEM_SHARED((CRNG + 8, AW), jnp.float32),
            pltpu.VMEM((MMW,), jnp.float32),
            pltpu.VMEM((MMW,), jnp.float32),
            pltpu.VMEM((DGW,), jnp.float32),
            pltpu.VMEM((LCAP,), jnp.int32),
            pltpu.VMEM((SCH,), jnp.int32),
            pltpu.VMEM((SCH,), jnp.int32),
            pltpu.VMEM((SCH,), jnp.int32),
            pltpu.VMEM((SCH,), jnp.int32),
            pltpu.VMEM((GC, AW), jnp.float32),
            pltpu.VMEM((GC, AW), jnp.float32),
            pltpu.VMEM((GC,), jnp.int32),
            pltpu.VMEM((GC,), jnp.int32),
            pltpu.VMEM((GC,), jnp.int32),
            pltpu.VMEM((GC,), jnp.int32),
            pltpu.VMEM((GC,), jnp.int32),
            pltpu.VMEM((GC,), jnp.int32),
            pltpu.SemaphoreType.DMA,
            pltpu.SemaphoreType.DMA,
            pltpu.SemaphoreType.DMA,
            pltpu.SemaphoreType.DMA,
            pltpu.SemaphoreType.DMA,
            pltpu.SemaphoreType.DMA,
            pltpu.SemaphoreType.DMA,
            pltpu.SemaphoreType.DMA,
        ],
    )
    return f(dst, src, xaug)


R = 1000
NB = N // R


def _dense_body(deg_ref, s_ref, ss_ref, mx_ref, mn_ref, x_ref,
                w1i_ref, w1a_ref, w1t_ref, b1_ref, w2_ref, b2_ref,
                w3_ref, b3_ref, g_ref, be_ref, out_ref, h3_scr, acc_scr):
    p = pl.program_id(0)
    i = pl.program_id(1)

    @pl.when(p == 0)
    def _phase0():
        deg = deg_ref[:, :]
        degs = jnp.maximum(deg, 1.0)
        has = deg > 0
        mean = s_ref[:, :] / degs
        meansq = ss_ref[:, :] / degs
        var = jnp.maximum(meansq - mean * mean, 0.0)
        std = jnp.sqrt(var + 1e-5)
        mx = jnp.where(has, mx_ref[:, :], 0.0)
        mn = jnp.where(has, mn_ref[:, :], 0.0)
        agg = jnp.concatenate([mean, mx, mn, std], axis=1)
        logd = jnp.log(deg + 1.0)
        ampf = logd / DELTA
        attf = jnp.where(deg > 0, DELTA / jnp.where(logd > 0, logd, 1.0), 0.0)
        hh = jnp.concatenate([agg, agg * ampf, agg * attf], axis=1)
        w1 = jnp.concatenate([w1i_ref[:, :], w1a_ref[:, :], w1t_ref[:, :]],
                             axis=0)
        h1 = jnp.maximum(
            jnp.dot(hh, w1, preferred_element_type=jnp.float32)
            + b1_ref[:, :], 0.0)
        h2 = jnp.maximum(
            jnp.dot(h1, w2_ref[:, :], preferred_element_type=jnp.float32)
            + b2_ref[:, :], 0.0)
        h3 = (jnp.dot(h2, w3_ref[:, :], preferred_element_type=jnp.float32)
              + b3_ref[:, :])
        h3_scr[pl.ds(i * R, R), :] = h3

        @pl.when(i == 0)
        def _init():
            acc_scr[:, :] = jnp.zeros_like(acc_scr)

        acc_scr[0:1, :] += jnp.sum(h3, axis=0, keepdims=True)
        acc_scr[1:2, :] += jnp.sum(h3 * h3, axis=0, keepdims=True)

    @pl.when(p == 1)
    def _phase1():
        mu = acc_scr[0:1, :] / float(N)
        var = acc_scr[1:2, :] / float(N) - mu * mu
        h3 = h3_scr[pl.ds(i * R, R), :]
        hn = (h3 - mu) / jnp.sqrt(var + 1e-5) * g_ref[:, :] + be_ref[:, :]
        out_ref[:, :] = jnp.maximum(hn, 0.0) + x_ref[:, :]


@functools.partial(jax.jit, static_argnames=("interpret",))
def _dense(deg, s, ss, mx, mn, x, W1, b1, W2, b2, W3, b3, gamma, beta,
           interpret=False):
    w1i = W1[0:512]
    w1a = W1[512:1024]
    w1t = W1[1024:1536]
    row = lambda r: pl.BlockSpec((R, r), lambda p, i: (i, 0))
    full = lambda a, b: pl.BlockSpec((a, b), lambda p, i: (0, 0))
    return pl.pallas_call(
        _dense_body,
        grid=(2, NB),
        in_specs=[
            row(1), row(F), row(F), row(F), row(F), row(F),
            full(512, F), full(512, F), full(512, F), full(1, F),
            full(F, F), full(1, F), full(F, F), full(1, F),
            full(1, F), full(1, F),
        ],
        out_specs=row(F),
        out_shape=jax.ShapeDtypeStruct((N, F), jnp.float32),
        scratch_shapes=[
            pltpu.VMEM((N, F), jnp.float32),
            pltpu.VMEM((2, F), jnp.float32),
        ],
        interpret=interpret,
    )(deg.reshape(N, 1), s, ss, mx, mn, x,
      w1i, w1a, w1t, b1.reshape(1, F), W2, b2.reshape(1, F),
      W3, b3.reshape(1, F), gamma.reshape(1, F), beta.reshape(1, F))


def kernel(x, edge_index, W1, b1, W2, b2, W3, b3, gamma, beta):
    src = edge_index[0]
    dst = edge_index[1]
    xc = jnp.stack([x[:, :HAF], x[:, HAF:]])
    xaug = jnp.concatenate([xc, xc * xc], axis=-1).reshape(2 * N, AW)

    sums, mxo, mno, dego = _sc_stats(dst, src, xaug)
    s = jnp.concatenate([sums[:N, :HAF], sums[NPAD:NPAD + N, :HAF]], axis=1)
    ss = jnp.concatenate([sums[:N, HAF:], sums[NPAD:NPAD + N, HAF:]], axis=1)
    mxr = mxo.reshape(2, NPAD, HAF)
    mnr = mno.reshape(2, NPAD, HAF)
    mx = jnp.concatenate([mxr[0, :N], mxr[1, :N]], axis=1)
    mn = jnp.concatenate([mnr[0, :N], mnr[1, :N]], axis=1)
    deg = dego.reshape(NPAD, 16)[:N, 0]
    return _dense(deg, s, ss, mx, mn, x, W1, b1, W2, b2, W3, b3, gamma, beta)
